# baseline (speedup 1.0000x reference)
import functools

import jax
import jax.numpy as jnp
from jax import lax
from jax.experimental import pallas as pl
from jax.experimental.pallas import tpu as pltpu
from jax.experimental.pallas import tpu_sc as plsc

N = 10000
D = 128
DH = D // 2
NC = 2
NS = 16
NW = NC * NS
C = 128
LANES = 16
NPAD = 10240
ZROWS = NPAD // NS

_MESH = plsc.VectorSubcoreMesh(
    core_axis_name="c", subcore_axis_name="s", num_cores=NC, num_subcores=NS)
_SC_PARAMS = pltpu.CompilerParams(needs_layout_passes=False,
                                  use_tc_tiling_on_sc=False)


_SLOW_CORE = 1
_SLOW_FRAC = 0.42


def _split_chunks(e_total):
    e_slow = int(e_total * _SLOW_FRAC)
    nch_slow = -(-e_slow // (NS * C))
    nch_slow = -(-nch_slow // 3) * 3
    e_fast = e_total - NS * nch_slow * C
    if e_fast < 0:
        e_fast = 0
    nch_fast = -(-e_fast // (NS * C))
    nch_fast = -(-nch_fast // 3) * 3
    return nch_slow, nch_fast


def _pack_shards(arr, e_total, nch_slow, nch_fast):
    nch = max(nch_slow, nch_fast)
    e_slow = NS * nch_slow * C
    a = arr[:e_slow].reshape(NS, nch_slow * C)
    a = jnp.pad(a, ((0, 0), (0, (nch - nch_slow) * C)))
    rest = arr[e_slow:]
    b = jnp.pad(rest, (0, NS * nch_fast * C - rest.shape[0]))
    b = b.reshape(NS, nch_fast * C)
    b = jnp.pad(b, ((0, 0), (0, (nch - nch_fast) * C)))
    if _SLOW_CORE == 0:
        packed = jnp.concatenate([a, b], axis=0)
    else:
        packed = jnp.concatenate([b, a], axis=0)
    return packed.reshape(NW, nch, C)


def _deg_call(dstp, ewp, nch):
    @functools.partial(
        pl.kernel,
        out_type=jax.ShapeDtypeStruct((NW, N), jnp.float32),
        mesh=_MESH,
        compiler_params=_SC_PARAMS,
        scratch_types=[
            pltpu.VMEM((nch, C), jnp.int32),
            pltpu.VMEM((nch, C), jnp.float32),
            pltpu.VMEM((N,), jnp.float32),
        ],
    )
    def deg_kernel(dst_hbm, ew_hbm, out_hbm, dst_v, ew_v, deg_v):
        c = lax.axis_index("c")
        s = lax.axis_index("s")
        wid = c * NS + s
        pltpu.sync_copy(dst_hbm.at[wid], dst_v)
        pltpu.sync_copy(ew_hbm.at[wid], ew_v)

        @pl.loop(0, N // LANES)
        def _zero(i):
            deg_v[pl.ds(i * LANES, LANES)] = jnp.zeros((LANES,), jnp.float32)

        lane = lax.iota(jnp.int32, LANES)

        @pl.loop(0, nch)
        def _chunk(j):
            for g in range(C // LANES):
                dstv = dst_v[j, pl.ds(g * LANES, LANES)]
                ewv = ew_v[j, pl.ds(g * LANES, LANES)]
                for l in range(LANES):
                    plsc.addupdate_scatter(deg_v, [dstv], ewv,
                                           mask=lane == l)

        pltpu.sync_copy(deg_v, out_hbm.at[wid])

    return deg_kernel(dstp, ewp)


def _edge_call(ya, yb, srcp, dstp, ewp, nch, nch_slow, nch_fast):
    out_sd = jax.ShapeDtypeStruct((NC, NPAD, DH), jnp.float32)

    @functools.partial(
        pl.kernel,
        out_type=[out_sd, out_sd],
        mesh=_MESH,
        compiler_params=_SC_PARAMS,
        scratch_types=[
            pltpu.VMEM((nch, C), jnp.int32),
            pltpu.VMEM((nch, C), jnp.int32),
            pltpu.VMEM((nch, C), jnp.float32),
            pltpu.VMEM((C, DH), jnp.float32),
            pltpu.VMEM((C, DH), jnp.float32),
            pltpu.VMEM((C, DH), jnp.float32),
            pltpu.VMEM_SHARED((NPAD, DH), jnp.float32),
            pltpu.SemaphoreType.DMA,
            pltpu.SemaphoreType.DMA,
            pltpu.SemaphoreType.DMA,
            pltpu.SemaphoreType.DMA,
            pltpu.SemaphoreType.DMA,
            pltpu.SemaphoreType.DMA,
        ],
    )
    def edge_kernel(ya_hbm, yb_hbm, src_hbm, dst_hbm, ew_hbm,
                    outa_hbm, outb_hbm,
                    src_v, dst_v, ew_v, b0, b1, b2, acc,
                    sg0, sg1, sg2, ss0, ss1, ss2):
        c = lax.axis_index("c")
        s = lax.axis_index("s")
        wid = c * NS + s
        pltpu.sync_copy(src_hbm.at[wid], src_v)
        pltpu.sync_copy(dst_hbm.at[wid], dst_v)
        pltpu.sync_copy(ew_hbm.at[wid], ew_v)
        base = s * ZROWS
        nch_c = jnp.where(c == _SLOW_CORE, nch_slow, nch_fast)

        bufs = (b0, b1, b2)
        gsems = (sg0, sg1, sg2)
        ssems = (ss0, ss1, ss2)

        for y_hbm, out_hbm in ((ya_hbm, outa_hbm), (yb_hbm, outb_hbm)):
            @pl.loop(0, C)
            def _zbuf(i):
                for g in range(DH // LANES):
                    b0[i, pl.ds(g * LANES, LANES)] = jnp.zeros(
                        (LANES,), jnp.float32)

            for r in range(0, ZROWS, C):
                rows = min(C, ZROWS - r)
                pltpu.sync_copy(b0.at[pl.ds(0, rows)],
                                acc.at[pl.ds(base + r, rows)])
            plsc.subcore_barrier()

            pltpu.async_copy(y_hbm.at[src_v.at[0]], b0, sg0)
            pltpu.async_copy(y_hbm.at[src_v.at[1]], b1, sg1)

            @pl.loop(0, nch_c, step=3)
            def _triple(j0):
                for b in range(3):
                    j = j0 + b
                    buf = bufs[b]
                    pltpu.make_async_copy(y_hbm.at[src_v.at[j]], buf,
                                          gsems[b]).wait()

                    j16 = jnp.broadcast_to(j, (LANES,)).astype(jnp.int32)

                    @plsc.parallel_loop(0, C, step=4, unroll=2)
                    def _edges(e0):
                        e16b = jnp.broadcast_to(e0, (LANES,)).astype(
                            jnp.int32)
                        for q in range(4):
                            w = plsc.load_gather(ew_v, [j16, e16b + q])
                            for g in range(DH // LANES):
                                sl = pl.ds(g * LANES, LANES)
                                buf[e0 + q, sl] = buf[e0 + q, sl] * w

                    pltpu.async_copy(buf, acc.at[dst_v.at[j]], ssems[b],
                                     add=True)

                    @pl.when(j >= 1)
                    def _wait_prev_scatter():
                        pltpu.make_async_copy(bufs[(b + 2) % 3],
                                              acc.at[dst_v.at[j - 1]],
                                              ssems[(b + 2) % 3]).wait()

                    @pl.when(j + 2 < nch_c)
                    def _next_gather():
                        pltpu.async_copy(y_hbm.at[src_v.at[j + 2]],
                                         bufs[(b + 2) % 3],
                                         gsems[(b + 2) % 3])

            pltpu.make_async_copy(bufs[2], acc.at[dst_v.at[nch_c - 1]],
                                  ssems[2]).wait()
            plsc.subcore_barrier()
            pltpu.sync_copy(acc.at[pl.ds(base, ZROWS)],
                            out_hbm.at[c].at[pl.ds(base, ZROWS)])

    return edge_kernel(ya, yb, srcp, dstp, ewp)


_ROWS_BLK = 1000
_GRID = N // _ROWS_BLK


def _tc_dinv(degp):
    def body(degp_ref, dinv_ref):
        deg = jnp.sum(degp_ref[...], axis=0) + 1.0
        dinv_ref[...] = lax.rsqrt(deg)[:, None]

    return pl.pallas_call(
        body,
        out_shape=jax.ShapeDtypeStruct((N, 1), jnp.float32),
    )(degp)


def _halved_out_specs():
    return (
        [
            pl.BlockSpec((_ROWS_BLK, DH), lambda i: (i, 0)),
            pl.BlockSpec((_ROWS_BLK, DH), lambda i: (i, 0)),
        ],
        [
            jax.ShapeDtypeStruct((N, DH), jnp.float32),
            jax.ShapeDtypeStruct((N, DH), jnp.float32),
        ],
    )


def _tc_first(x, w, dinv):
    def body(x_ref, w_ref, dinv_ref, ya_ref, yb_ref):
        y = dinv_ref[...] * jnp.dot(x_ref[...], w_ref[...],
                                    preferred_element_type=jnp.float32)
        ya_ref[...] = y[:, :DH]
        yb_ref[...] = y[:, DH:]

    out_specs, out_shape = _halved_out_specs()
    return pl.pallas_call(
        body,
        grid=(_GRID,),
        in_specs=[
            pl.BlockSpec((_ROWS_BLK, D), lambda i: (i, 0)),
            pl.BlockSpec((D, D), lambda i: (0, 0)),
            pl.BlockSpec((_ROWS_BLK, 1), lambda i: (i, 0)),
        ],
        out_specs=out_specs,
        out_shape=out_shape,
    )(x, w, dinv)


def _part_in_specs():
    return [
        pl.BlockSpec((NC, _ROWS_BLK, DH), lambda i: (0, i, 0)),
        pl.BlockSpec((NC, _ROWS_BLK, DH), lambda i: (0, i, 0)),
        pl.BlockSpec((_ROWS_BLK, DH), lambda i: (i, 0)),
        pl.BlockSpec((_ROWS_BLK, DH), lambda i: (i, 0)),
        pl.BlockSpec((_ROWS_BLK, 1), lambda i: (i, 0)),
        pl.BlockSpec((1, D), lambda i: (0, 0)),
    ]


def _combine(pa_ref, pb_ref, ya_ref, yb_ref, dinv_ref, b_ref):
    pa = pa_ref[0] + pa_ref[1] + ya_ref[...]
    pb = pb_ref[0] + pb_ref[1] + yb_ref[...]
    pre = jnp.concatenate([pa, pb], axis=1)
    return dinv_ref[...] * pre + b_ref[...]


def _tc_mid(pa, pb, ya, yb, dinv, b, w_next):
    def body(pa_ref, pb_ref, ya_ref, yb_ref, dinv_ref, b_ref, w_ref,
             oa_ref, ob_ref):
        h = jnp.maximum(_combine(pa_ref, pb_ref, ya_ref, yb_ref,
                                 dinv_ref, b_ref), 0.0)
        y = dinv_ref[...] * jnp.dot(h, w_ref[...],
                                    preferred_element_type=jnp.float32)
        oa_ref[...] = y[:, :DH]
        ob_ref[...] = y[:, DH:]

    out_specs, out_shape = _halved_out_specs()
    return pl.pallas_call(
        body,
        grid=(_GRID,),
        in_specs=_part_in_specs() + [pl.BlockSpec((D, D), lambda i: (0, 0))],
        out_specs=out_specs,
        out_shape=out_shape,
    )(pa, pb, ya, yb, dinv, b, w_next)


def _tc_last(pa, pb, ya, yb, dinv, b):
    def body(pa_ref, pb_ref, ya_ref, yb_ref, dinv_ref, b_ref, o_ref):
        o_ref[...] = _combine(pa_ref, pb_ref, ya_ref, yb_ref,
                              dinv_ref, b_ref)

    return pl.pallas_call(
        body,
        grid=(_GRID,),
        in_specs=_part_in_specs(),
        out_specs=pl.BlockSpec((_ROWS_BLK, D), lambda i: (i, 0)),
        out_shape=jax.ShapeDtypeStruct((N, D), jnp.float32),
    )(pa, pb, ya, yb, dinv, b)


def kernel(x, edge_index, edge_weights, W1, b1, W2, b2, W3, b3):
    e_total = edge_index.shape[1]
    nch_slow, nch_fast = _split_chunks(e_total)
    nch = max(nch_slow, nch_fast)

    src = _pack_shards(edge_index[0], e_total, nch_slow, nch_fast)
    dst = _pack_shards(edge_index[1], e_total, nch_slow, nch_fast)
    ew = _pack_shards(edge_weights, e_total, nch_slow, nch_fast)

    degp = _deg_call(dst, ew, nch)
    dinv = _tc_dinv(degp)
    b1r = b1.reshape(1, D)
    b2r = b2.reshape(1, D)
    b3r = b3.reshape(1, D)

    y1a, y1b = _tc_first(x, W1, dinv)
    p1a, p1b = _edge_call(y1a, y1b, src, dst, ew, nch, nch_slow, nch_fast)
    y2a, y2b = _tc_mid(p1a, p1b, y1a, y1b, dinv, b1r, W2)
    p2a, p2b = _edge_call(y2a, y2b, src, dst, ew, nch, nch_slow, nch_fast)
    y3a, y3b = _tc_mid(p2a, p2b, y2a, y2b, dinv, b2r, W3)
    p3a, p3b = _edge_call(y3a, y3b, src, dst, ew, nch, nch_slow, nch_fast)
    return _tc_last(p3a, p3b, y3a, y3b, dinv, b3r)


# speedup vs baseline: 1.0600x; 1.0600x over previous
import functools

import jax
import jax.numpy as jnp
from jax import lax
from jax.experimental import pallas as pl
from jax.experimental.pallas import tpu as pltpu
from jax.experimental.pallas import tpu_sc as plsc

N = 10000
D = 128
DH = D // 2
NC = 2
NS = 16
NW = NC * NS
C = 128
LANES = 16
NPAD = 10240
ZROWS = NPAD // NS

_MESH = plsc.VectorSubcoreMesh(
    core_axis_name="c", subcore_axis_name="s", num_cores=NC, num_subcores=NS)
_SC_PARAMS = pltpu.CompilerParams(needs_layout_passes=False,
                                  use_tc_tiling_on_sc=False)


_SLOW_CORE = 1
_SLOW_FRAC = 0.46


def _split_chunks(e_total):
    e_slow = int(e_total * _SLOW_FRAC)
    nch_slow = -(-e_slow // (NS * C))
    nch_slow = -(-nch_slow // 3) * 3
    e_fast = e_total - NS * nch_slow * C
    if e_fast < 0:
        e_fast = 0
    nch_fast = -(-e_fast // (NS * C))
    nch_fast = -(-nch_fast // 3) * 3
    return nch_slow, nch_fast


def _pack_shards(arr, e_total, nch_slow, nch_fast):
    nch = max(nch_slow, nch_fast)
    e_slow = NS * nch_slow * C
    a = arr[:e_slow].reshape(NS, nch_slow * C)
    a = jnp.pad(a, ((0, 0), (0, (nch - nch_slow) * C)))
    rest = arr[e_slow:]
    b = jnp.pad(rest, (0, NS * nch_fast * C - rest.shape[0]))
    b = b.reshape(NS, nch_fast * C)
    b = jnp.pad(b, ((0, 0), (0, (nch - nch_fast) * C)))
    if _SLOW_CORE == 0:
        packed = jnp.concatenate([a, b], axis=0)
    else:
        packed = jnp.concatenate([b, a], axis=0)
    return packed.reshape(NW, nch, C)


def _deg_call(dstp, ewp, nch):
    @functools.partial(
        pl.kernel,
        out_type=jax.ShapeDtypeStruct((NW, N), jnp.float32),
        mesh=_MESH,
        compiler_params=_SC_PARAMS,
        scratch_types=[
            pltpu.VMEM((nch, C), jnp.int32),
            pltpu.VMEM((nch, C), jnp.float32),
            pltpu.VMEM((N,), jnp.float32),
        ],
    )
    def deg_kernel(dst_hbm, ew_hbm, out_hbm, dst_v, ew_v, deg_v):
        c = lax.axis_index("c")
        s = lax.axis_index("s")
        wid = c * NS + s
        pltpu.sync_copy(dst_hbm.at[wid], dst_v)
        pltpu.sync_copy(ew_hbm.at[wid], ew_v)

        @pl.loop(0, N // LANES)
        def _zero(i):
            deg_v[pl.ds(i * LANES, LANES)] = jnp.zeros((LANES,), jnp.float32)

        lane = lax.iota(jnp.int32, LANES)

        @pl.loop(0, nch)
        def _chunk(j):
            for g in range(C // LANES):
                dstv = dst_v[j, pl.ds(g * LANES, LANES)]
                ewv = ew_v[j, pl.ds(g * LANES, LANES)]
                for l in range(LANES):
                    plsc.addupdate_scatter(deg_v, [dstv], ewv,
                                           mask=lane == l)

        pltpu.sync_copy(deg_v, out_hbm.at[wid])

    return deg_kernel(dstp, ewp)


def _edge_call(ya, yb, srcp, dstp, ewp, nch, nch_slow, nch_fast):
    out_sd = jax.ShapeDtypeStruct((NC, NPAD, DH), jnp.float32)

    @functools.partial(
        pl.kernel,
        out_type=[out_sd, out_sd],
        mesh=_MESH,
        compiler_params=_SC_PARAMS,
        scratch_types=[
            pltpu.VMEM((nch, C), jnp.int32),
            pltpu.VMEM((nch, C), jnp.int32),
            pltpu.VMEM((nch, C), jnp.float32),
            pltpu.VMEM((C, DH), jnp.float32),
            pltpu.VMEM((C, DH), jnp.float32),
            pltpu.VMEM((C, DH), jnp.float32),
            pltpu.VMEM_SHARED((NPAD, DH), jnp.float32),
            pltpu.SemaphoreType.DMA,
            pltpu.SemaphoreType.DMA,
            pltpu.SemaphoreType.DMA,
            pltpu.SemaphoreType.DMA,
            pltpu.SemaphoreType.DMA,
            pltpu.SemaphoreType.DMA,
        ],
    )
    def edge_kernel(ya_hbm, yb_hbm, src_hbm, dst_hbm, ew_hbm,
                    outa_hbm, outb_hbm,
                    src_v, dst_v, ew_v, b0, b1, b2, acc,
                    sg0, sg1, sg2, ss0, ss1, ss2):
        c = lax.axis_index("c")
        s = lax.axis_index("s")
        wid = c * NS + s
        pltpu.sync_copy(src_hbm.at[wid], src_v)
        pltpu.sync_copy(dst_hbm.at[wid], dst_v)
        pltpu.sync_copy(ew_hbm.at[wid], ew_v)
        base = s * ZROWS
        nch_c = jnp.where(c == _SLOW_CORE, nch_slow, nch_fast)

        bufs = (b0, b1, b2)
        gsems = (sg0, sg1, sg2)
        ssems = (ss0, ss1, ss2)

        for y_hbm, out_hbm in ((ya_hbm, outa_hbm), (yb_hbm, outb_hbm)):
            @pl.loop(0, C)
            def _zbuf(i):
                for g in range(DH // LANES):
                    b0[i, pl.ds(g * LANES, LANES)] = jnp.zeros(
                        (LANES,), jnp.float32)

            for r in range(0, ZROWS, C):
                rows = min(C, ZROWS - r)
                pltpu.sync_copy(b0.at[pl.ds(0, rows)],
                                acc.at[pl.ds(base + r, rows)])
            plsc.subcore_barrier()

            pltpu.async_copy(y_hbm.at[src_v.at[0]], b0, sg0)
            pltpu.async_copy(y_hbm.at[src_v.at[1]], b1, sg1)

            @pl.loop(0, nch_c, step=3)
            def _triple(j0):
                for b in range(3):
                    j = j0 + b
                    buf = bufs[b]
                    pltpu.make_async_copy(y_hbm.at[src_v.at[j]], buf,
                                          gsems[b]).wait()

                    j16 = jnp.broadcast_to(j, (LANES,)).astype(jnp.int32)

                    @plsc.parallel_loop(0, C, step=4, unroll=2)
                    def _edges(e0):
                        e16b = jnp.broadcast_to(e0, (LANES,)).astype(
                            jnp.int32)
                        for q in range(4):
                            w = plsc.load_gather(ew_v, [j16, e16b + q])
                            for g in range(DH // LANES):
                                sl = pl.ds(g * LANES, LANES)
                                buf[e0 + q, sl] = buf[e0 + q, sl] * w

                    pltpu.async_copy(buf, acc.at[dst_v.at[j]], ssems[b],
                                     add=True)

                    @pl.when(j >= 1)
                    def _wait_prev_scatter():
                        pltpu.make_async_copy(bufs[(b + 2) % 3],
                                              acc.at[dst_v.at[j - 1]],
                                              ssems[(b + 2) % 3]).wait()

                    @pl.when(j + 2 < nch_c)
                    def _next_gather():
                        pltpu.async_copy(y_hbm.at[src_v.at[j + 2]],
                                         bufs[(b + 2) % 3],
                                         gsems[(b + 2) % 3])

            pltpu.make_async_copy(bufs[2], acc.at[dst_v.at[nch_c - 1]],
                                  ssems[2]).wait()
            plsc.subcore_barrier()
            pltpu.sync_copy(acc.at[pl.ds(base, ZROWS)],
                            out_hbm.at[c].at[pl.ds(base, ZROWS)])

    return edge_kernel(ya, yb, srcp, dstp, ewp)


_ROWS_BLK = 1000
_GRID = N // _ROWS_BLK


def _tc_dinv(degp):
    def body(degp_ref, dinv_ref):
        deg = jnp.sum(degp_ref[...], axis=0) + 1.0
        dinv_ref[...] = lax.rsqrt(deg)[:, None]

    return pl.pallas_call(
        body,
        out_shape=jax.ShapeDtypeStruct((N, 1), jnp.float32),
    )(degp)


def _halved_out_specs():
    return (
        [
            pl.BlockSpec((_ROWS_BLK, DH), lambda i: (i, 0)),
            pl.BlockSpec((_ROWS_BLK, DH), lambda i: (i, 0)),
        ],
        [
            jax.ShapeDtypeStruct((N, DH), jnp.float32),
            jax.ShapeDtypeStruct((N, DH), jnp.float32),
        ],
    )


def _tc_first(x, w, dinv):
    def body(x_ref, w_ref, dinv_ref, ya_ref, yb_ref):
        y = dinv_ref[...] * jnp.dot(x_ref[...], w_ref[...],
                                    preferred_element_type=jnp.float32)
        ya_ref[...] = y[:, :DH]
        yb_ref[...] = y[:, DH:]

    out_specs, out_shape = _halved_out_specs()
    return pl.pallas_call(
        body,
        grid=(_GRID,),
        in_specs=[
            pl.BlockSpec((_ROWS_BLK, D), lambda i: (i, 0)),
            pl.BlockSpec((D, D), lambda i: (0, 0)),
            pl.BlockSpec((_ROWS_BLK, 1), lambda i: (i, 0)),
        ],
        out_specs=out_specs,
        out_shape=out_shape,
    )(x, w, dinv)


def _part_in_specs():
    return [
        pl.BlockSpec((NC, _ROWS_BLK, DH), lambda i: (0, i, 0)),
        pl.BlockSpec((NC, _ROWS_BLK, DH), lambda i: (0, i, 0)),
        pl.BlockSpec((_ROWS_BLK, DH), lambda i: (i, 0)),
        pl.BlockSpec((_ROWS_BLK, DH), lambda i: (i, 0)),
        pl.BlockSpec((_ROWS_BLK, 1), lambda i: (i, 0)),
        pl.BlockSpec((1, D), lambda i: (0, 0)),
    ]


def _combine(pa_ref, pb_ref, ya_ref, yb_ref, dinv_ref, b_ref):
    pa = pa_ref[0] + pa_ref[1] + ya_ref[...]
    pb = pb_ref[0] + pb_ref[1] + yb_ref[...]
    pre = jnp.concatenate([pa, pb], axis=1)
    return dinv_ref[...] * pre + b_ref[...]


def _tc_mid(pa, pb, ya, yb, dinv, b, w_next):
    def body(pa_ref, pb_ref, ya_ref, yb_ref, dinv_ref, b_ref, w_ref,
             oa_ref, ob_ref):
        h = jnp.maximum(_combine(pa_ref, pb_ref, ya_ref, yb_ref,
                                 dinv_ref, b_ref), 0.0)
        y = dinv_ref[...] * jnp.dot(h, w_ref[...],
                                    preferred_element_type=jnp.float32)
        oa_ref[...] = y[:, :DH]
        ob_ref[...] = y[:, DH:]

    out_specs, out_shape = _halved_out_specs()
    return pl.pallas_call(
        body,
        grid=(_GRID,),
        in_specs=_part_in_specs() + [pl.BlockSpec((D, D), lambda i: (0, 0))],
        out_specs=out_specs,
        out_shape=out_shape,
    )(pa, pb, ya, yb, dinv, b, w_next)


def _tc_last(pa, pb, ya, yb, dinv, b):
    def body(pa_ref, pb_ref, ya_ref, yb_ref, dinv_ref, b_ref, o_ref):
        o_ref[...] = _combine(pa_ref, pb_ref, ya_ref, yb_ref,
                              dinv_ref, b_ref)

    return pl.pallas_call(
        body,
        grid=(_GRID,),
        in_specs=_part_in_specs(),
        out_specs=pl.BlockSpec((_ROWS_BLK, D), lambda i: (i, 0)),
        out_shape=jax.ShapeDtypeStruct((N, D), jnp.float32),
    )(pa, pb, ya, yb, dinv, b)


def kernel(x, edge_index, edge_weights, W1, b1, W2, b2, W3, b3):
    e_total = edge_index.shape[1]
    nch_slow, nch_fast = _split_chunks(e_total)
    nch = max(nch_slow, nch_fast)

    src = _pack_shards(edge_index[0], e_total, nch_slow, nch_fast)
    dst = _pack_shards(edge_index[1], e_total, nch_slow, nch_fast)
    ew = _pack_shards(edge_weights, e_total, nch_slow, nch_fast)

    degp = _deg_call(dst, ew, nch)
    dinv = _tc_dinv(degp)
    b1r = b1.reshape(1, D)
    b2r = b2.reshape(1, D)
    b3r = b3.reshape(1, D)

    y1a, y1b = _tc_first(x, W1, dinv)
    p1a, p1b = _edge_call(y1a, y1b, src, dst, ew, nch, nch_slow, nch_fast)
    y2a, y2b = _tc_mid(p1a, p1b, y1a, y1b, dinv, b1r, W2)
    p2a, p2b = _edge_call(y2a, y2b, src, dst, ew, nch, nch_slow, nch_fast)
    y3a, y3b = _tc_mid(p2a, p2b, y2a, y2b, dinv, b2r, W3)
    p3a, p3b = _edge_call(y3a, y3b, src, dst, ew, nch, nch_slow, nch_fast)
    return _tc_last(p3a, p3b, y3a, y3b, dinv, b3r)


# speedup vs baseline: 1.1155x; 1.0524x over previous
import functools

import jax
import jax.numpy as jnp
from jax import lax
from jax.experimental import pallas as pl
from jax.experimental.pallas import tpu as pltpu
from jax.experimental.pallas import tpu_sc as plsc

N = 10000
D = 128
DH = D // 2
NC = 2
NS = 16
NW = NC * NS
C = 128
LANES = 16
NPAD = 10240
ZROWS = NPAD // NS

_MESH = plsc.VectorSubcoreMesh(
    core_axis_name="c", subcore_axis_name="s", num_cores=NC, num_subcores=NS)
_SC_PARAMS = pltpu.CompilerParams(needs_layout_passes=False,
                                  use_tc_tiling_on_sc=False)


_SLOW_CORE = 1
_SLOW_FRAC = 0.49


def _split_chunks(e_total):
    e_slow = int(e_total * _SLOW_FRAC)
    nch_slow = -(-e_slow // (NS * C))
    nch_slow = -(-nch_slow // 3) * 3
    e_fast = e_total - NS * nch_slow * C
    if e_fast < 0:
        e_fast = 0
    nch_fast = -(-e_fast // (NS * C))
    nch_fast = -(-nch_fast // 3) * 3
    return nch_slow, nch_fast


def _pack_shards(arr, e_total, nch_slow, nch_fast):
    nch = max(nch_slow, nch_fast)
    e_slow = NS * nch_slow * C
    a = arr[:e_slow].reshape(NS, nch_slow * C)
    a = jnp.pad(a, ((0, 0), (0, (nch - nch_slow) * C)))
    rest = arr[e_slow:]
    b = jnp.pad(rest, (0, NS * nch_fast * C - rest.shape[0]))
    b = b.reshape(NS, nch_fast * C)
    b = jnp.pad(b, ((0, 0), (0, (nch - nch_fast) * C)))
    if _SLOW_CORE == 0:
        packed = jnp.concatenate([a, b], axis=0)
    else:
        packed = jnp.concatenate([b, a], axis=0)
    return packed.reshape(NW, nch, C)


def _deg_call(dstp, ewp, nch):
    @functools.partial(
        pl.kernel,
        out_type=jax.ShapeDtypeStruct((NW, N), jnp.float32),
        mesh=_MESH,
        compiler_params=_SC_PARAMS,
        scratch_types=[
            pltpu.VMEM((nch, C), jnp.int32),
            pltpu.VMEM((nch, C), jnp.float32),
            pltpu.VMEM((N,), jnp.float32),
        ],
    )
    def deg_kernel(dst_hbm, ew_hbm, out_hbm, dst_v, ew_v, deg_v):
        c = lax.axis_index("c")
        s = lax.axis_index("s")
        wid = c * NS + s
        pltpu.sync_copy(dst_hbm.at[wid], dst_v)
        pltpu.sync_copy(ew_hbm.at[wid], ew_v)

        @pl.loop(0, N // LANES)
        def _zero(i):
            deg_v[pl.ds(i * LANES, LANES)] = jnp.zeros((LANES,), jnp.float32)

        lane = lax.iota(jnp.int32, LANES)

        @pl.loop(0, nch)
        def _chunk(j):
            for g in range(C // LANES):
                dstv = dst_v[j, pl.ds(g * LANES, LANES)]
                ewv = ew_v[j, pl.ds(g * LANES, LANES)]
                for l in range(LANES):
                    plsc.addupdate_scatter(deg_v, [dstv], ewv,
                                           mask=lane == l)

        pltpu.sync_copy(deg_v, out_hbm.at[wid])

    return deg_kernel(dstp, ewp)


def _edge_call(ya, yb, srcp, dstp, ewp, nch, nch_slow, nch_fast):
    out_sd = jax.ShapeDtypeStruct((NC, NPAD, DH), jnp.float32)

    @functools.partial(
        pl.kernel,
        out_type=[out_sd, out_sd],
        mesh=_MESH,
        compiler_params=_SC_PARAMS,
        scratch_types=[
            pltpu.VMEM((nch, C), jnp.int32),
            pltpu.VMEM((nch, C), jnp.int32),
            pltpu.VMEM((nch, C), jnp.float32),
            pltpu.VMEM((C, DH), jnp.float32),
            pltpu.VMEM((C, DH), jnp.float32),
            pltpu.VMEM((C, DH), jnp.float32),
            pltpu.VMEM_SHARED((NPAD, DH), jnp.float32),
            pltpu.SemaphoreType.DMA,
            pltpu.SemaphoreType.DMA,
            pltpu.SemaphoreType.DMA,
            pltpu.SemaphoreType.DMA,
            pltpu.SemaphoreType.DMA,
            pltpu.SemaphoreType.DMA,
        ],
    )
    def edge_kernel(ya_hbm, yb_hbm, src_hbm, dst_hbm, ew_hbm,
                    outa_hbm, outb_hbm,
                    src_v, dst_v, ew_v, b0, b1, b2, acc,
                    sg0, sg1, sg2, ss0, ss1, ss2):
        c = lax.axis_index("c")
        s = lax.axis_index("s")
        wid = c * NS + s
        pltpu.sync_copy(src_hbm.at[wid], src_v)
        pltpu.sync_copy(dst_hbm.at[wid], dst_v)
        pltpu.sync_copy(ew_hbm.at[wid], ew_v)
        base = s * ZROWS
        nch_c = jnp.where(c == _SLOW_CORE, nch_slow, nch_fast)

        bufs = (b0, b1, b2)
        gsems = (sg0, sg1, sg2)
        ssems = (ss0, ss1, ss2)

        for y_hbm, out_hbm in ((ya_hbm, outa_hbm), (yb_hbm, outb_hbm)):
            @pl.loop(0, C)
            def _zbuf(i):
                for g in range(DH // LANES):
                    b0[i, pl.ds(g * LANES, LANES)] = jnp.zeros(
                        (LANES,), jnp.float32)

            for r in range(0, ZROWS, C):
                rows = min(C, ZROWS - r)
                pltpu.sync_copy(b0.at[pl.ds(0, rows)],
                                acc.at[pl.ds(base + r, rows)])
            plsc.subcore_barrier()

            pltpu.async_copy(y_hbm.at[src_v.at[0]], b0, sg0)
            pltpu.async_copy(y_hbm.at[src_v.at[1]], b1, sg1)

            @pl.loop(0, nch_c, step=3)
            def _triple(j0):
                for b in range(3):
                    j = j0 + b
                    buf = bufs[b]
                    pltpu.make_async_copy(y_hbm.at[src_v.at[j]], buf,
                                          gsems[b]).wait()

                    j16 = jnp.broadcast_to(j, (LANES,)).astype(jnp.int32)

                    @plsc.parallel_loop(0, C, step=4, unroll=2)
                    def _edges(e0):
                        e16b = jnp.broadcast_to(e0, (LANES,)).astype(
                            jnp.int32)
                        for q in range(4):
                            w = plsc.load_gather(ew_v, [j16, e16b + q])
                            for g in range(DH // LANES):
                                sl = pl.ds(g * LANES, LANES)
                                buf[e0 + q, sl] = buf[e0 + q, sl] * w

                    pltpu.async_copy(buf, acc.at[dst_v.at[j]], ssems[b],
                                     add=True)

                    @pl.when(j >= 1)
                    def _wait_prev_scatter():
                        pltpu.make_async_copy(bufs[(b + 2) % 3],
                                              acc.at[dst_v.at[j - 1]],
                                              ssems[(b + 2) % 3]).wait()

                    @pl.when(j + 2 < nch_c)
                    def _next_gather():
                        pltpu.async_copy(y_hbm.at[src_v.at[j + 2]],
                                         bufs[(b + 2) % 3],
                                         gsems[(b + 2) % 3])

            pltpu.make_async_copy(bufs[2], acc.at[dst_v.at[nch_c - 1]],
                                  ssems[2]).wait()
            plsc.subcore_barrier()
            pltpu.sync_copy(acc.at[pl.ds(base, ZROWS)],
                            out_hbm.at[c].at[pl.ds(base, ZROWS)])

    return edge_kernel(ya, yb, srcp, dstp, ewp)


_ROWS_BLK = 1000
_GRID = N // _ROWS_BLK


def _tc_dinv(degp):
    def body(degp_ref, dinv_ref):
        deg = jnp.sum(degp_ref[...], axis=0) + 1.0
        dinv_ref[...] = lax.rsqrt(deg)[:, None]

    return pl.pallas_call(
        body,
        out_shape=jax.ShapeDtypeStruct((N, 1), jnp.float32),
    )(degp)


def _halved_out_specs():
    return (
        [
            pl.BlockSpec((_ROWS_BLK, DH), lambda i: (i, 0)),
            pl.BlockSpec((_ROWS_BLK, DH), lambda i: (i, 0)),
        ],
        [
            jax.ShapeDtypeStruct((N, DH), jnp.float32),
            jax.ShapeDtypeStruct((N, DH), jnp.float32),
        ],
    )


def _tc_first(x, w, dinv):
    def body(x_ref, w_ref, dinv_ref, ya_ref, yb_ref):
        y = dinv_ref[...] * jnp.dot(x_ref[...], w_ref[...],
                                    preferred_element_type=jnp.float32)
        ya_ref[...] = y[:, :DH]
        yb_ref[...] = y[:, DH:]

    out_specs, out_shape = _halved_out_specs()
    return pl.pallas_call(
        body,
        grid=(_GRID,),
        in_specs=[
            pl.BlockSpec((_ROWS_BLK, D), lambda i: (i, 0)),
            pl.BlockSpec((D, D), lambda i: (0, 0)),
            pl.BlockSpec((_ROWS_BLK, 1), lambda i: (i, 0)),
        ],
        out_specs=out_specs,
        out_shape=out_shape,
    )(x, w, dinv)


def _part_in_specs():
    return [
        pl.BlockSpec((NC, _ROWS_BLK, DH), lambda i: (0, i, 0)),
        pl.BlockSpec((NC, _ROWS_BLK, DH), lambda i: (0, i, 0)),
        pl.BlockSpec((_ROWS_BLK, DH), lambda i: (i, 0)),
        pl.BlockSpec((_ROWS_BLK, DH), lambda i: (i, 0)),
        pl.BlockSpec((_ROWS_BLK, 1), lambda i: (i, 0)),
        pl.BlockSpec((1, D), lambda i: (0, 0)),
    ]


def _combine(pa_ref, pb_ref, ya_ref, yb_ref, dinv_ref, b_ref):
    pa = pa_ref[0] + pa_ref[1] + ya_ref[...]
    pb = pb_ref[0] + pb_ref[1] + yb_ref[...]
    pre = jnp.concatenate([pa, pb], axis=1)
    return dinv_ref[...] * pre + b_ref[...]


def _tc_mid(pa, pb, ya, yb, dinv, b, w_next):
    def body(pa_ref, pb_ref, ya_ref, yb_ref, dinv_ref, b_ref, w_ref,
             oa_ref, ob_ref):
        h = jnp.maximum(_combine(pa_ref, pb_ref, ya_ref, yb_ref,
                                 dinv_ref, b_ref), 0.0)
        y = dinv_ref[...] * jnp.dot(h, w_ref[...],
                                    preferred_element_type=jnp.float32)
        oa_ref[...] = y[:, :DH]
        ob_ref[...] = y[:, DH:]

    out_specs, out_shape = _halved_out_specs()
    return pl.pallas_call(
        body,
        grid=(_GRID,),
        in_specs=_part_in_specs() + [pl.BlockSpec((D, D), lambda i: (0, 0))],
        out_specs=out_specs,
        out_shape=out_shape,
    )(pa, pb, ya, yb, dinv, b, w_next)


def _tc_last(pa, pb, ya, yb, dinv, b):
    def body(pa_ref, pb_ref, ya_ref, yb_ref, dinv_ref, b_ref, o_ref):
        o_ref[...] = _combine(pa_ref, pb_ref, ya_ref, yb_ref,
                              dinv_ref, b_ref)

    return pl.pallas_call(
        body,
        grid=(_GRID,),
        in_specs=_part_in_specs(),
        out_specs=pl.BlockSpec((_ROWS_BLK, D), lambda i: (i, 0)),
        out_shape=jax.ShapeDtypeStruct((N, D), jnp.float32),
    )(pa, pb, ya, yb, dinv, b)


def kernel(x, edge_index, edge_weights, W1, b1, W2, b2, W3, b3):
    e_total = edge_index.shape[1]
    nch_slow, nch_fast = _split_chunks(e_total)
    nch = max(nch_slow, nch_fast)

    src = _pack_shards(edge_index[0], e_total, nch_slow, nch_fast)
    dst = _pack_shards(edge_index[1], e_total, nch_slow, nch_fast)
    ew = _pack_shards(edge_weights, e_total, nch_slow, nch_fast)

    degp = _deg_call(dst, ew, nch)
    dinv = _tc_dinv(degp)
    b1r = b1.reshape(1, D)
    b2r = b2.reshape(1, D)
    b3r = b3.reshape(1, D)

    y1a, y1b = _tc_first(x, W1, dinv)
    p1a, p1b = _edge_call(y1a, y1b, src, dst, ew, nch, nch_slow, nch_fast)
    y2a, y2b = _tc_mid(p1a, p1b, y1a, y1b, dinv, b1r, W2)
    p2a, p2b = _edge_call(y2a, y2b, src, dst, ew, nch, nch_slow, nch_fast)
    y3a, y3b = _tc_mid(p2a, p2b, y2a, y2b, dinv, b2r, W3)
    p3a, p3b = _edge_call(y3a, y3b, src, dst, ew, nch, nch_slow, nch_fast)
    return _tc_last(p3a, p3b, y3a, y3b, dinv, b3r)
